# baseline (speedup 1.0000x reference)
import functools

import jax
import jax.numpy as jnp
from jax import lax
from jax.experimental import pallas as pl
from jax.experimental.pallas import tpu as pltpu
from jax.experimental.pallas import tpu_sc as plsc

B = 16384
D = 32
H = 64

_NC = 2
_NS = 16
_NW = _NC * _NS
_BPW = B // _NW
_CHUNK = 128
_NCHUNK = _BPW // _CHUNK


@functools.partial(
    pl.kernel,
    out_type=(
        jax.ShapeDtypeStruct((B, D), jnp.float32),
        jax.ShapeDtypeStruct((B, D), jnp.float32),
    ),
    mesh=plsc.VectorSubcoreMesh(core_axis_name="c", subcore_axis_name="s"),
    scratch_types=[
        pltpu.VMEM((_NCHUNK, _CHUNK), jnp.int32),
        pltpu.VMEM((_NCHUNK, _CHUNK), jnp.int32),
        pltpu.VMEM((_BPW, D), jnp.float32),
        pltpu.VMEM((_BPW, D), jnp.float32),
        pltpu.SemaphoreType.DMA,
        pltpu.SemaphoreType.DMA,
    ],
    compiler_params=pltpu.CompilerParams(use_tc_tiling_on_sc=False),
)
def _sc_gather(uid2d, iid2d, utab, itab, out_u, out_i,
               idx_u, idx_i, rows_u, rows_i, sem_u, sem_i):
    wid = lax.axis_index("s") * _NC + lax.axis_index("c")
    base = wid * _BPW
    pltpu.sync_copy(uid2d.at[pl.ds(wid * _NCHUNK, _NCHUNK)], idx_u)
    pltpu.sync_copy(iid2d.at[pl.ds(wid * _NCHUNK, _NCHUNK)], idx_i)
    copies = []
    for j in range(_NCHUNK):
        copies.append(pltpu.async_copy(
            utab.at[idx_u.at[j]], rows_u.at[pl.ds(j * _CHUNK, _CHUNK)], sem_u))
        copies.append(pltpu.async_copy(
            itab.at[idx_i.at[j]], rows_i.at[pl.ds(j * _CHUNK, _CHUNK)], sem_i))
    for c in copies:
        c.wait()
    pltpu.sync_copy(rows_u, out_u.at[pl.ds(base, _BPW)])
    pltpu.sync_copy(rows_i, out_i.at[pl.ds(base, _BPW)])


_BLK = 2048


def _mlp_body(u_ref, i_ref, w_ref, p_ref, out_ref):
    h = jnp.dot(u_ref[...], w_ref[0:D, :], preferred_element_type=jnp.float32)
    h = h + jnp.dot(i_ref[...], w_ref[D:2 * D, :],
                    preferred_element_type=jnp.float32)
    h = jnp.maximum(h + p_ref[0:1, :], 0.0)
    out_ref[...] = (jnp.sum(h * p_ref[1:2, :], axis=1, keepdims=True)
                    + p_ref[2:3, 0:1])


_mlp = pl.pallas_call(
    _mlp_body,
    grid=(B // _BLK,),
    in_specs=[
        pl.BlockSpec((_BLK, D), lambda i: (i, 0)),
        pl.BlockSpec((_BLK, D), lambda i: (i, 0)),
        pl.BlockSpec((2 * D, H), lambda i: (0, 0)),
        pl.BlockSpec((8, H), lambda i: (0, 0)),
    ],
    out_specs=pl.BlockSpec((_BLK, 1), lambda i: (i, 0)),
    out_shape=jax.ShapeDtypeStruct((B, 1), jnp.float32),
)


def kernel(user_ids, item_ids, user_table, item_table, W1, b1, Wf, bf):
    uid2d = user_ids.reshape(B // _CHUNK, _CHUNK)
    iid2d = item_ids.reshape(B // _CHUNK, _CHUNK)
    u_rows, i_rows = _sc_gather(uid2d, iid2d, user_table, item_table)
    w = W1.T
    params = jnp.concatenate(
        [b1.reshape(1, H), Wf.reshape(1, H),
         jnp.broadcast_to(bf.reshape(1, 1), (1, H)),
         jnp.zeros((5, H), jnp.float32)], axis=0)
    return _mlp(u_rows, i_rows, w, params)
